# baseline (speedup 1.0000x reference)
import jax
import jax.numpy as jnp
from jax import lax
from jax.experimental import pallas as pl
from jax.experimental.pallas import tpu as pltpu

_NC = 80
_MAX_DET = 300
_NMS_IOU = 0.7
_CONF_THRES = 0.001
_LANES = 128
_ROWS = 264
_APAD = _ROWS * _LANES
_NK = 384
_IBIG = 2147483647


def _detect_kernel(x_ref, o_ref, s_ref, cls_ref, cand_ref):
    sc = x_ref[0, 4:, :, :]
    smax = jnp.max(sc, axis=0)
    ci = lax.broadcasted_iota(jnp.int32, (_NC, _ROWS, _LANES), 0)
    cls = jnp.min(jnp.where(sc == smax[None], ci, _IBIG), axis=0)
    cls_ref[...] = cls.astype(jnp.float32)
    s_ref[...] = smax

    lT = lax.broadcasted_iota(jnp.int32, (1, _NK), 1)
    ohl = lax.broadcasted_iota(jnp.int32, (1, _LANES), 1)
    lR = lax.broadcasted_iota(jnp.int32, (1, _ROWS), 1)

    colmax = jnp.max(smax, axis=1, keepdims=True)
    eyeR = (lax.broadcasted_iota(jnp.int32, (_ROWS, _ROWS), 0)
            == lax.broadcasted_iota(jnp.int32, (_ROWS, _ROWS), 1))
    rm0 = jnp.sum(jnp.where(eyeR, jnp.broadcast_to(colmax, (_ROWS, _ROWS)), 0.0),
                  axis=0, keepdims=True)

    zrow = jnp.zeros((1, _NK), jnp.float32)

    def topk_body(i, carry):
        rm, cxT, cyT, wT, hT, cfT, clT = carry
        mx = jnp.max(rm)
        r = jnp.min(jnp.where(rm == mx, lR, _IBIG))
        srow = s_ref[pl.dslice(r, 1), :]
        c = jnp.min(jnp.where(srow == mx, ohl, _IBIG))
        oh = ohl == c
        row0 = x_ref[0, 0, pl.dslice(r, 1), :]
        row1 = x_ref[0, 1, pl.dslice(r, 1), :]
        row2 = x_ref[0, 2, pl.dslice(r, 1), :]
        row3 = x_ref[0, 3, pl.dslice(r, 1), :]
        rowc = cls_ref[pl.dslice(r, 1), :]
        cx = jnp.sum(jnp.where(oh, row0, 0.0))
        cy = jnp.sum(jnp.where(oh, row1, 0.0))
        w = jnp.sum(jnp.where(oh, row2, 0.0))
        h = jnp.sum(jnp.where(oh, row3, 0.0))
        cl = jnp.sum(jnp.where(oh, rowc, 0.0))
        srow = jnp.where(oh, -2.0, srow)
        s_ref[pl.dslice(r, 1), :] = srow
        rm = jnp.where(lR == r, jnp.max(srow), rm)
        upd = lT == i
        cxT = jnp.where(upd, cx, cxT)
        cyT = jnp.where(upd, cy, cyT)
        wT = jnp.where(upd, w, wT)
        hT = jnp.where(upd, h, hT)
        cfT = jnp.where(upd, mx, cfT)
        clT = jnp.where(upd, cl, clT)
        return rm, cxT, cyT, wT, hT, cfT, clT

    carry0 = (rm0, zrow, zrow, zrow, zrow, zrow, zrow)
    _, cxT, cyT, wT, hT, cfT, clT = lax.fori_loop(
        0, _MAX_DET, topk_body, carry0)

    x1T = cxT - wT * 0.5
    y1T = cyT - hT * 0.5
    x2T = cxT + wT * 0.5
    y2T = cyT + hT * 0.5
    areaT = (x2T - x1T) * (y2T - y1T)

    ri = lax.broadcasted_iota(jnp.int32, (_NK, _NK), 0)
    cj = lax.broadcasted_iota(jnp.int32, (_NK, _NK), 1)
    eye = ri == cj

    def tocol(row):
        return jnp.sum(jnp.where(eye, jnp.broadcast_to(row, (_NK, _NK)), 0.0),
                       axis=1, keepdims=True)

    x1C, y1C, x2C, y2C, areaC = (tocol(v) for v in (x1T, y1T, x2T, y2T, areaT))
    iw = jnp.maximum(jnp.minimum(x2C, x2T) - jnp.maximum(x1C, x1T), 0.0)
    ih = jnp.maximum(jnp.minimum(y2C, y2T) - jnp.maximum(y1C, y1T), 0.0)
    inter = iw * ih
    iou = inter / (areaC + areaT - inter + 1e-7)
    cand_ref[...] = jnp.where((iou > _NMS_IOU) & (cj > ri), 1.0, 0.0)

    def nms_body(i, keep):
        ki = jnp.sum(jnp.where(lT == i, keep, 0.0))
        kif = jnp.where(ki > 0.5, 1.0, 0.0)
        row = cand_ref[pl.dslice(i, 1), :]
        return keep * (1.0 - row * kif)

    keep = lax.fori_loop(0, _MAX_DET, nms_body, jnp.ones((1, _NK), jnp.float32))

    valid = (keep > 0.5) & (cfT > _CONF_THRES)
    cfo = jnp.where(valid, cfT, 0.0)

    cols = [tocol(v) for v in (cxT, cyT, wT, hT, cfo, clT)]
    l6 = lax.broadcasted_iota(jnp.int32, (_NK, 6), 1)
    out = jnp.zeros((_NK, 6), jnp.float32)
    for k in range(6):
        out = out + jnp.where(l6 == k, cols[k], 0.0)
    o_ref[0] = out[:_MAX_DET, :]


def kernel(raw):
    B, C, A = raw.shape
    pad = _APAD - A
    boxes = raw[:, :4, :]
    scores = raw[:, 4:, :]
    boxes_p = jnp.pad(boxes, ((0, 0), (0, 0), (0, pad)))
    scores_p = jnp.pad(scores, ((0, 0), (0, 0), (0, pad)), constant_values=-1.0)
    xp = jnp.concatenate([boxes_p, scores_p], axis=1).reshape(B, C, _ROWS, _LANES)

    return pl.pallas_call(
        _detect_kernel,
        grid=(B,),
        in_specs=[pl.BlockSpec((1, C, _ROWS, _LANES), lambda b: (b, 0, 0, 0))],
        out_specs=pl.BlockSpec((1, _MAX_DET, 6), lambda b: (b, 0, 0)),
        out_shape=jax.ShapeDtypeStruct((B, _MAX_DET, 6), jnp.float32),
        scratch_shapes=[pltpu.VMEM((_ROWS, _LANES), jnp.float32),
                        pltpu.VMEM((_ROWS, _LANES), jnp.float32),
                        pltpu.VMEM((_NK, _NK), jnp.float32)],
    )(xp)


# speedup vs baseline: 1.1770x; 1.1770x over previous
import jax
import jax.numpy as jnp
from jax import lax
from jax.experimental import pallas as pl
from jax.experimental.pallas import tpu as pltpu

_NC = 80
_MAX_DET = 300
_NMS_IOU = 0.7
_CONF_THRES = 0.001
_LANES = 128
_ROWS = 264
_APAD = _ROWS * _LANES
_NK = 384
_IBIG = 2147483647


def _detect_kernel(x_ref, o_ref, s_ref, cls_ref):
    sc = x_ref[0, 4:, :, :]
    smax = jnp.max(sc, axis=0)
    ci = lax.broadcasted_iota(jnp.int32, (_NC, _ROWS, _LANES), 0)
    cls = jnp.min(jnp.where(sc == smax[None], ci, _IBIG), axis=0)
    cls_ref[...] = cls.astype(jnp.float32)
    s_ref[...] = smax

    lT = lax.broadcasted_iota(jnp.int32, (1, _NK), 1)
    ohl = lax.broadcasted_iota(jnp.int32, (1, _LANES), 1)
    lR = lax.broadcasted_iota(jnp.int32, (1, _ROWS), 1)

    colmax = jnp.max(smax, axis=1, keepdims=True)
    eyeR = (lax.broadcasted_iota(jnp.int32, (_ROWS, _ROWS), 0)
            == lax.broadcasted_iota(jnp.int32, (_ROWS, _ROWS), 1))
    rm0 = jnp.sum(jnp.where(eyeR, jnp.broadcast_to(colmax, (_ROWS, _ROWS)), 0.0),
                  axis=0, keepdims=True)

    zirow = jnp.zeros((1, _NK), jnp.int32)
    zfrow = jnp.zeros((1, _NK), jnp.float32)

    def topk_body(i, carry):
        rm, rT, cT, cfT = carry
        mx = jnp.max(rm)
        r = jnp.min(jnp.where(rm == mx, lR, _IBIG))
        srow = s_ref[pl.dslice(r, 1), :]
        c = jnp.min(jnp.where(srow == mx, ohl, _IBIG))
        srow = jnp.where(ohl == c, -2.0, srow)
        s_ref[pl.dslice(r, 1), :] = srow
        rm = jnp.where(lR == r, jnp.max(srow), rm)
        upd = lT == i
        rT = jnp.where(upd, r, rT)
        cT = jnp.where(upd, c, cT)
        cfT = jnp.where(upd, mx, cfT)
        return rm, rT, cT, cfT

    _, rT, cT, cfT = lax.fori_loop(
        0, _MAX_DET, topk_body, (rm0, zirow, zirow, zfrow))

    ri = lax.broadcasted_iota(jnp.int32, (_NK, _NK), 0)
    cj = lax.broadcasted_iota(jnp.int32, (_NK, _NK), 1)
    eye = ri == cj

    def tocol_f(row):
        return jnp.sum(jnp.where(eye, jnp.broadcast_to(row, (_NK, _NK)), 0.0),
                       axis=1, keepdims=True)

    def tocol_i(row):
        return jnp.sum(jnp.where(eye, jnp.broadcast_to(row, (_NK, _NK)), 0),
                       axis=1, keepdims=True)

    rC = tocol_i(rT)
    cC = tocol_i(cT)
    cfC = tocol_f(cfT)

    rsel = jnp.where(
        lax.broadcasted_iota(jnp.int32, (_NK, _ROWS), 1) == rC, 1.0, 0.0)
    lane1h = jnp.where(
        lax.broadcasted_iota(jnp.int32, (_NK, _LANES), 1) == cC, 1.0, 0.0)

    def gather_plane(plane):
        rows = jnp.dot(rsel, plane, preferred_element_type=jnp.float32)
        return jnp.sum(rows * lane1h, axis=1, keepdims=True)

    cxC = gather_plane(x_ref[0, 0])
    cyC = gather_plane(x_ref[0, 1])
    wC = gather_plane(x_ref[0, 2])
    hC = gather_plane(x_ref[0, 3])
    clC = gather_plane(cls_ref[...])

    x1C = cxC - wC * 0.5
    y1C = cyC - hC * 0.5
    x2C = cxC + wC * 0.5
    y2C = cyC + hC * 0.5
    areaC = (x2C - x1C) * (y2C - y1C)

    def torow(col):
        return jnp.sum(jnp.where(eye, jnp.broadcast_to(col, (_NK, _NK)), 0.0),
                       axis=0, keepdims=True)

    x1T, y1T, x2T, y2T, areaT = (torow(v) for v in (x1C, y1C, x2C, y2C, areaC))
    iw = jnp.maximum(jnp.minimum(x2C, x2T) - jnp.maximum(x1C, x1T), 0.0)
    ih = jnp.maximum(jnp.minimum(y2C, y2T) - jnp.maximum(y1C, y1T), 0.0)
    inter = iw * ih
    iou = inter / (areaC + areaT - inter + 1e-7)
    cand = jnp.where((iou > _NMS_IOU) & (cj > ri), 1.0, 0.0)

    def nms_cond(carry):
        _, changed = carry
        return changed

    def nms_step(carry):
        keepC, _ = carry
        supT = jnp.max(cand * keepC, axis=0, keepdims=True)
        newT = jnp.where(supT > 0.5, 0.0, 1.0)
        newC = tocol_f(newT)
        return (newC, jnp.any(newC != keepC))

    keepC, _ = lax.while_loop(
        nms_cond, nms_step, (jnp.ones((_NK, 1), jnp.float32), True))

    valid = (keepC > 0.5) & (cfC > _CONF_THRES)
    cfoC = jnp.where(valid, cfC, 0.0)

    l6 = lax.broadcasted_iota(jnp.int32, (_NK, 6), 1)
    out = jnp.zeros((_NK, 6), jnp.float32)
    for k, col in enumerate((cxC, cyC, wC, hC, cfoC, clC)):
        out = out + jnp.where(l6 == k, col, 0.0)
    o_ref[0] = out[:_MAX_DET, :]


def kernel(raw):
    B, C, A = raw.shape
    pad = _APAD - A
    boxes = raw[:, :4, :]
    scores = raw[:, 4:, :]
    boxes_p = jnp.pad(boxes, ((0, 0), (0, 0), (0, pad)))
    scores_p = jnp.pad(scores, ((0, 0), (0, 0), (0, pad)), constant_values=-1.0)
    xp = jnp.concatenate([boxes_p, scores_p], axis=1).reshape(B, C, _ROWS, _LANES)

    return pl.pallas_call(
        _detect_kernel,
        grid=(B,),
        in_specs=[pl.BlockSpec((1, C, _ROWS, _LANES), lambda b: (b, 0, 0, 0))],
        out_specs=pl.BlockSpec((1, _MAX_DET, 6), lambda b: (b, 0, 0)),
        out_shape=jax.ShapeDtypeStruct((B, _MAX_DET, 6), jnp.float32),
        scratch_shapes=[pltpu.VMEM((_ROWS, _LANES), jnp.float32),
                        pltpu.VMEM((_ROWS, _LANES), jnp.float32)],
    )(xp)


# speedup vs baseline: 1.3297x; 1.1298x over previous
import jax
import jax.numpy as jnp
from jax import lax
from jax.experimental import pallas as pl
from jax.experimental.pallas import tpu as pltpu

_NC = 80
_MAX_DET = 300
_NMS_IOU = 0.7
_CONF_THRES = 0.001
_LANES = 128
_ROWS = 264
_APAD = _ROWS * _LANES
_NK = 384
_IBIG = 2147483647
_B = 16


def _classmax_kernel(x_ref, smax_ref, cls_ref):
    sc = x_ref[0]
    smax = jnp.max(sc, axis=0)
    ci = lax.broadcasted_iota(jnp.int32, (_NC, _ROWS, _LANES), 0)
    cls = jnp.min(jnp.where(sc == smax[None], ci, _IBIG), axis=0)
    smax_ref[0] = smax
    cls_ref[0] = cls.astype(jnp.float32)


def _select_kernel(smax_ref, cls_ref, box_ref, o_ref, s_ref):
    s_ref[...] = smax_ref[...]
    nb = smax_ref.shape[0]

    lT = lax.broadcasted_iota(jnp.int32, (1, _NK), 1)
    ohl = lax.broadcasted_iota(jnp.int32, (1, _LANES), 1)
    lR = lax.broadcasted_iota(jnp.int32, (1, _ROWS), 1)

    rms = [jnp.max(s_ref[b], axis=1)[None, :] for b in range(nb)]
    zi = jnp.zeros((1, _NK), jnp.int32)
    zf = jnp.zeros((1, _NK), jnp.float32)

    def topk_body(i, carry):
        rms, rTs, cTs, cfTs = carry
        out_rm, out_r, out_c, out_cf = [], [], [], []
        upd = lT == i
        for b in range(nb):
            rm = rms[b]
            mx = jnp.max(rm)
            r = jnp.min(jnp.where(rm == mx, lR, _IBIG))
            srow = s_ref[b, pl.dslice(r, 1), :]
            c = jnp.min(jnp.where(srow == mx, ohl, _IBIG))
            srow = jnp.where(ohl == c, -2.0, srow)
            s_ref[b, pl.dslice(r, 1), :] = srow
            out_rm.append(jnp.where(lR == r, jnp.max(srow), rm))
            out_r.append(jnp.where(upd, r, rTs[b]))
            out_c.append(jnp.where(upd, c, cTs[b]))
            out_cf.append(jnp.where(upd, mx, cfTs[b]))
        return tuple(out_rm), tuple(out_r), tuple(out_c), tuple(out_cf)

    carry0 = (tuple(rms), (zi,) * nb, (zi,) * nb, (zf,) * nb)
    _, rTs, cTs, cfTs = lax.fori_loop(0, _MAX_DET, topk_body, carry0)

    ri = lax.broadcasted_iota(jnp.int32, (_NK, _NK), 0)
    cj = lax.broadcasted_iota(jnp.int32, (_NK, _NK), 1)
    eye = ri == cj

    def tocol_f(row):
        return jnp.sum(jnp.where(eye, jnp.broadcast_to(row, (_NK, _NK)), 0.0),
                       axis=1, keepdims=True)

    def tocol_i(row):
        return jnp.sum(jnp.where(eye, jnp.broadcast_to(row, (_NK, _NK)), 0),
                       axis=1, keepdims=True)

    iotaNR = lax.broadcasted_iota(jnp.int32, (_NK, _ROWS), 1)
    iotaNL = lax.broadcasted_iota(jnp.int32, (_NK, _LANES), 1)
    l6 = lax.broadcasted_iota(jnp.int32, (_NK, 6), 1)

    cands = []
    keeps = []
    percols = []
    for b in range(nb):
        rC = tocol_i(rTs[b])
        cC = tocol_i(cTs[b])
        cfC = tocol_f(cfTs[b])
        rsel = jnp.where(iotaNR == rC, 1.0, 0.0)
        lane1h = jnp.where(iotaNL == cC, 1.0, 0.0)

        def gather(plane, prec):
            rows = jax.lax.dot(rsel, plane, precision=prec,
                               preferred_element_type=jnp.float32)
            return jnp.sum(rows * lane1h, axis=1, keepdims=True)

        hi = jax.lax.Precision.HIGHEST
        cxC = gather(box_ref[b, 0], hi)
        cyC = gather(box_ref[b, 1], hi)
        wC = gather(box_ref[b, 2], hi)
        hC = gather(box_ref[b, 3], hi)
        clC = gather(cls_ref[b], jax.lax.Precision.DEFAULT)

        x1C = cxC - wC * 0.5
        y1C = cyC - hC * 0.5
        x2C = cxC + wC * 0.5
        y2C = cyC + hC * 0.5
        areaC = (x2C - x1C) * (y2C - y1C)

        def torow(col):
            return jnp.sum(
                jnp.where(eye, jnp.broadcast_to(col, (_NK, _NK)), 0.0),
                axis=0, keepdims=True)

        x1T, y1T, x2T, y2T, areaT = (
            torow(v) for v in (x1C, y1C, x2C, y2C, areaC))
        iw = jnp.maximum(jnp.minimum(x2C, x2T) - jnp.maximum(x1C, x1T), 0.0)
        ih = jnp.maximum(jnp.minimum(y2C, y2T) - jnp.maximum(y1C, y1T), 0.0)
        inter = iw * ih
        iou = inter / (areaC + areaT - inter + 1e-7)
        cands.append(jnp.where((iou > _NMS_IOU) & (cj > ri), 1.0, 0.0))
        keeps.append(jnp.ones((1, _NK), jnp.float32))
        percols.append((cxC, cyC, wC, hC, cfC, clC))

    def nms_cond(carry):
        return carry[1]

    def nms_step(carry):
        keeps, _ = carry
        new = []
        changed = False
        for b in range(nb):
            sup = jax.lax.dot(keeps[b], cands[b],
                              preferred_element_type=jnp.float32)
            nk = jnp.where(sup > 0.5, 0.0, 1.0)
            changed = changed | jnp.any(nk != keeps[b])
            new.append(nk)
        return tuple(new), changed

    keeps, _ = lax.while_loop(nms_cond, nms_step, (tuple(keeps), True))

    for b in range(nb):
        cxC, cyC, wC, hC, cfC, clC = percols[b]
        keepC = tocol_f(keeps[b])
        valid = (keepC > 0.5) & (cfC > _CONF_THRES)
        cfoC = jnp.where(valid, cfC, 0.0)
        out = jnp.zeros((_NK, 6), jnp.float32)
        for k, col in enumerate((cxC, cyC, wC, hC, cfoC, clC)):
            out = out + jnp.where(l6 == k, col, 0.0)
        o_ref[b] = out[:_MAX_DET, :]


def kernel(raw):
    B, C, A = raw.shape
    pad = _APAD - A
    boxes = raw[:, :4, :]
    scores = raw[:, 4:, :]
    boxes_p = jnp.pad(boxes, ((0, 0), (0, 0), (0, pad))).reshape(
        B, 4, _ROWS, _LANES)
    scores_p = jnp.pad(scores, ((0, 0), (0, 0), (0, pad)),
                       constant_values=-1.0).reshape(B, _NC, _ROWS, _LANES)

    smax, cls = pl.pallas_call(
        _classmax_kernel,
        grid=(B,),
        in_specs=[pl.BlockSpec((1, _NC, _ROWS, _LANES), lambda b: (b, 0, 0, 0))],
        out_specs=[pl.BlockSpec((1, _ROWS, _LANES), lambda b: (b, 0, 0)),
                   pl.BlockSpec((1, _ROWS, _LANES), lambda b: (b, 0, 0))],
        out_shape=[jax.ShapeDtypeStruct((B, _ROWS, _LANES), jnp.float32),
                   jax.ShapeDtypeStruct((B, _ROWS, _LANES), jnp.float32)],
    )(scores_p)

    return pl.pallas_call(
        _select_kernel,
        out_shape=jax.ShapeDtypeStruct((B, _MAX_DET, 6), jnp.float32),
        scratch_shapes=[pltpu.VMEM((B, _ROWS, _LANES), jnp.float32)],
    )(smax, cls, boxes_p)


# speedup vs baseline: 1.3302x; 1.0004x over previous
import jax
import jax.numpy as jnp
from jax import lax
from jax.experimental import pallas as pl
from jax.experimental.pallas import tpu as pltpu

_NC = 80
_MAX_DET = 300
_NMS_IOU = 0.7
_CONF_THRES = 0.001
_LANES = 128
_ROWS = 264
_APAD = _ROWS * _LANES
_NK = 384
_IBIG = 2147483647
_B = 16


def _classmax_kernel(x_ref, smax_ref, cls_ref):
    sc = x_ref[0]
    smax = jnp.max(sc, axis=0)
    ci = lax.broadcasted_iota(jnp.int32, (_NC, _ROWS, _LANES), 0)
    cls = jnp.min(jnp.where(sc == smax[None], ci, _IBIG), axis=0)
    smax_ref[0] = smax
    cls_ref[0] = cls.astype(jnp.float32)


def _select_kernel(smax_ref, cls_ref, box_ref, o_ref, *s_refs):
    nb = smax_ref.shape[0]
    for b in range(nb):
        s_refs[b][...] = smax_ref[b]

    lT = lax.broadcasted_iota(jnp.int32, (1, _NK), 1)
    ohl = lax.broadcasted_iota(jnp.int32, (1, _LANES), 1)
    lR = lax.broadcasted_iota(jnp.int32, (1, _ROWS), 1)

    rms = [jnp.max(s_refs[b][...], axis=1)[None, :] for b in range(nb)]
    zi = jnp.zeros((1, _NK), jnp.int32)
    zf = jnp.zeros((1, _NK), jnp.float32)

    def topk_body(i, carry):
        rms, rTs, cTs, cfTs = carry
        out_rm, out_r, out_c, out_cf = [], [], [], []
        upd = lT == i
        for b in range(nb):
            rm = rms[b]
            mx = jnp.max(rm)
            r = jnp.min(jnp.where(rm == mx, lR, _IBIG))
            srow = s_refs[b][pl.dslice(r, 1), :]
            c = jnp.min(jnp.where(srow == mx, ohl, _IBIG))
            srow = jnp.where(ohl == c, -2.0, srow)
            s_refs[b][pl.dslice(r, 1), :] = srow
            out_rm.append(jnp.where(lR == r, jnp.max(srow), rm))
            out_r.append(jnp.where(upd, r, rTs[b]))
            out_c.append(jnp.where(upd, c, cTs[b]))
            out_cf.append(jnp.where(upd, mx, cfTs[b]))
        return tuple(out_rm), tuple(out_r), tuple(out_c), tuple(out_cf)

    carry0 = (tuple(rms), (zi,) * nb, (zi,) * nb, (zf,) * nb)
    _, rTs, cTs, cfTs = lax.fori_loop(0, _MAX_DET, topk_body, carry0)

    ri = lax.broadcasted_iota(jnp.int32, (_NK, _NK), 0)
    cj = lax.broadcasted_iota(jnp.int32, (_NK, _NK), 1)
    eye = ri == cj

    def tocol_f(row):
        return jnp.sum(jnp.where(eye, jnp.broadcast_to(row, (_NK, _NK)), 0.0),
                       axis=1, keepdims=True)

    def tocol_i(row):
        return jnp.sum(jnp.where(eye, jnp.broadcast_to(row, (_NK, _NK)), 0),
                       axis=1, keepdims=True)

    iotaNR = lax.broadcasted_iota(jnp.int32, (_NK, _ROWS), 1)
    iotaNL = lax.broadcasted_iota(jnp.int32, (_NK, _LANES), 1)
    l6 = lax.broadcasted_iota(jnp.int32, (_NK, 6), 1)

    cands = []
    keeps = []
    percols = []
    for b in range(nb):
        rC = tocol_i(rTs[b])
        cC = tocol_i(cTs[b])
        cfC = tocol_f(cfTs[b])
        rsel = jnp.where(iotaNR == rC, 1.0, 0.0)
        lane1h = jnp.where(iotaNL == cC, 1.0, 0.0)

        def gather(plane, prec):
            rows = jax.lax.dot(rsel, plane, precision=prec,
                               preferred_element_type=jnp.float32)
            return jnp.sum(rows * lane1h, axis=1, keepdims=True)

        hi = jax.lax.Precision.HIGHEST
        cxC = gather(box_ref[b, 0], hi)
        cyC = gather(box_ref[b, 1], hi)
        wC = gather(box_ref[b, 2], hi)
        hC = gather(box_ref[b, 3], hi)
        clC = gather(cls_ref[b], jax.lax.Precision.DEFAULT)

        x1C = cxC - wC * 0.5
        y1C = cyC - hC * 0.5
        x2C = cxC + wC * 0.5
        y2C = cyC + hC * 0.5
        areaC = (x2C - x1C) * (y2C - y1C)

        def torow(col):
            return jnp.sum(
                jnp.where(eye, jnp.broadcast_to(col, (_NK, _NK)), 0.0),
                axis=0, keepdims=True)

        x1T, y1T, x2T, y2T, areaT = (
            torow(v) for v in (x1C, y1C, x2C, y2C, areaC))
        iw = jnp.maximum(jnp.minimum(x2C, x2T) - jnp.maximum(x1C, x1T), 0.0)
        ih = jnp.maximum(jnp.minimum(y2C, y2T) - jnp.maximum(y1C, y1T), 0.0)
        inter = iw * ih
        iou = inter / (areaC + areaT - inter + 1e-7)
        cands.append(jnp.where((iou > _NMS_IOU) & (cj > ri), 1.0, 0.0))
        keeps.append(jnp.ones((1, _NK), jnp.float32))
        percols.append((cxC, cyC, wC, hC, cfC, clC))

    def nms_cond(carry):
        return carry[1]

    def nms_step(carry):
        keeps, _ = carry
        new = []
        changed = False
        for b in range(nb):
            sup = jax.lax.dot(keeps[b], cands[b],
                              preferred_element_type=jnp.float32)
            nk = jnp.where(sup > 0.5, 0.0, 1.0)
            changed = changed | jnp.any(nk != keeps[b])
            new.append(nk)
        return tuple(new), changed

    keeps, _ = lax.while_loop(nms_cond, nms_step, (tuple(keeps), True))

    for b in range(nb):
        cxC, cyC, wC, hC, cfC, clC = percols[b]
        keepC = tocol_f(keeps[b])
        valid = (keepC > 0.5) & (cfC > _CONF_THRES)
        cfoC = jnp.where(valid, cfC, 0.0)
        out = jnp.zeros((_NK, 6), jnp.float32)
        for k, col in enumerate((cxC, cyC, wC, hC, cfoC, clC)):
            out = out + jnp.where(l6 == k, col, 0.0)
        o_ref[b] = out[:_MAX_DET, :]


def kernel(raw):
    B, C, A = raw.shape
    pad = _APAD - A
    boxes = raw[:, :4, :]
    scores = raw[:, 4:, :]
    boxes_p = jnp.pad(boxes, ((0, 0), (0, 0), (0, pad))).reshape(
        B, 4, _ROWS, _LANES)
    scores_p = jnp.pad(scores, ((0, 0), (0, 0), (0, pad)),
                       constant_values=-1.0).reshape(B, _NC, _ROWS, _LANES)

    smax, cls = pl.pallas_call(
        _classmax_kernel,
        grid=(B,),
        in_specs=[pl.BlockSpec((1, _NC, _ROWS, _LANES), lambda b: (b, 0, 0, 0))],
        out_specs=[pl.BlockSpec((1, _ROWS, _LANES), lambda b: (b, 0, 0)),
                   pl.BlockSpec((1, _ROWS, _LANES), lambda b: (b, 0, 0))],
        out_shape=[jax.ShapeDtypeStruct((B, _ROWS, _LANES), jnp.float32),
                   jax.ShapeDtypeStruct((B, _ROWS, _LANES), jnp.float32)],
    )(scores_p)

    return pl.pallas_call(
        _select_kernel,
        out_shape=jax.ShapeDtypeStruct((B, _MAX_DET, 6), jnp.float32),
        scratch_shapes=[pltpu.VMEM((_ROWS, _LANES), jnp.float32)
                        for _ in range(B)],
    )(smax, cls, boxes_p)
